# baseline (speedup 1.0000x reference)
import functools

import jax
import jax.numpy as jnp
from jax import lax
from jax.experimental import pallas as pl
from jax.experimental.pallas import tpu as pltpu
from jax.experimental.pallas import tpu_sc as plsc

_D = 256
_N = 8192
_T = 2304
_BETA = 0.25

_NB = 512
_TB = 768


def _argmin_body(e_ref, xm2t_ref, minval_ref, minidx_ref):
    n = pl.program_id(1)
    e = e_ref[...]
    s = jnp.dot(e, xm2t_ref[...], preferred_element_type=jnp.float32)
    esq = jnp.sum(e * e, axis=1, keepdims=True)
    dist = s + esq
    bmin = jnp.min(dist, axis=0, keepdims=True)
    row = lax.broadcasted_iota(jnp.int32, dist.shape, 0) + n * _NB
    bidx = jnp.min(jnp.where(dist == bmin, row, jnp.int32(2**30)),
                   axis=0, keepdims=True)

    @pl.when(n == 0)
    def _():
        minval_ref[...] = bmin
        minidx_ref[...] = bidx

    @pl.when(n > 0)
    def _():
        cur = minval_ref[...]
        take = bmin < cur
        minval_ref[...] = jnp.where(take, bmin, cur)
        minidx_ref[...] = jnp.where(take, bidx, minidx_ref[...])


def _vq_argmin(embedding, xm2t):
    return pl.pallas_call(
        _argmin_body,
        grid=(_T // _TB, _N // _NB),
        in_specs=[
            pl.BlockSpec((_NB, _D), lambda t, n: (n, 0)),
            pl.BlockSpec((_D, _TB), lambda t, n: (0, t)),
        ],
        out_specs=[
            pl.BlockSpec((1, _TB), lambda t, n: (0, t)),
            pl.BlockSpec((1, _TB), lambda t, n: (0, t)),
        ],
        out_shape=[
            jax.ShapeDtypeStruct((1, _T), jnp.float32),
            jax.ShapeDtypeStruct((1, _T), jnp.int32),
        ],
    )(embedding, xm2t)


def _sc_gather(table, idx):
    info = plsc.get_sparse_core_info()
    nc, ns = info.num_cores, info.num_subcores
    nw = nc * ns
    b_per_w = _T // nw
    mesh = plsc.VectorSubcoreMesh(core_axis_name="c", subcore_axis_name="s")

    @functools.partial(
        pl.kernel,
        mesh=mesh,
        out_type=jax.ShapeDtypeStruct((_T, _D), jnp.float32),
        scratch_types=[
            pltpu.VMEM((b_per_w,), jnp.int32),
            pltpu.VMEM((b_per_w, _D), jnp.float32),
            pltpu.SemaphoreType.DMA,
        ],
    )
    def k(table_hbm, idx_hbm, out_hbm, idx_v, rows_v, sem):
        wid = lax.axis_index("s") * nc + lax.axis_index("c")
        base = wid * b_per_w
        pltpu.sync_copy(idx_hbm.at[pl.ds(base, b_per_w)], idx_v)
        pltpu.async_copy(table_hbm.at[idx_v], rows_v, sem).wait()
        pltpu.sync_copy(rows_v, out_hbm.at[pl.ds(base, b_per_w)])

    return k(table, idx)


def _mlp_body(x_ref, z_ref, minval_ref, w1_ref, b1_ref, w2_ref, b2_ref,
              zout_ref, loss_ref):
    t = pl.program_id(0)
    x = x_ref[...]
    z = z_ref[...]
    r = x - z
    h = jnp.maximum(
        jnp.dot(r, w1_ref[...], preferred_element_type=jnp.float32)
        + b1_ref[...], 0.0)
    zout_ref[...] = (z + jnp.dot(h, w2_ref[...],
                                 preferred_element_type=jnp.float32)
                     + b2_ref[...])
    psum = (jnp.sum(minval_ref[...], keepdims=True)
            + jnp.sum(x * x, keepdims=True))
    prev = jnp.where(t == 0, jnp.zeros_like(psum), loss_ref[...])
    tot = prev + psum
    nblk = pl.num_programs(0)
    loss_ref[...] = jnp.where(t == nblk - 1, tot * (_BETA / _T), tot)


def _mlp(x2, z2, minval, w1, b1, w2, b2):
    return pl.pallas_call(
        _mlp_body,
        grid=(_T // _TB,),
        in_specs=[
            pl.BlockSpec((_TB, _D), lambda t: (t, 0)),
            pl.BlockSpec((_TB, _D), lambda t: (t, 0)),
            pl.BlockSpec((1, _TB), lambda t: (0, t)),
            pl.BlockSpec((_D, _D), lambda t: (0, 0)),
            pl.BlockSpec((1, _D), lambda t: (0, 0)),
            pl.BlockSpec((_D, _D), lambda t: (0, 0)),
            pl.BlockSpec((1, _D), lambda t: (0, 0)),
        ],
        out_specs=[
            pl.BlockSpec((_TB, _D), lambda t: (t, 0)),
            pl.BlockSpec((1, 1), lambda t: (0, 0)),
        ],
        out_shape=[
            jax.ShapeDtypeStruct((_T, _D), jnp.float32),
            jax.ShapeDtypeStruct((1, 1), jnp.float32),
        ],
    )(x2, z2, minval, w1, b1, w2, b2)


def kernel(x, embedding, W1, b1, W2, b2):
    x2 = x.reshape(_T, _D)
    xm2t = (-2.0 * x2).T
    minval, minidx = _vq_argmin(embedding, xm2t)
    z2 = _sc_gather(embedding, minidx.reshape(_T))
    zout, loss = _mlp(x2, z2, minval, W1, b1.reshape(1, _D),
                      W2, b2.reshape(1, _D))
    return zout.reshape(x.shape), loss[0, 0]
